# Initial kernel scaffold; baseline (speedup 1.0000x reference)
import jax
import jax.numpy as jnp
from jax.experimental import pallas as pl


def kernel(x, edge_index, W1_mu, W1_rho, b1_mu, b1_rho, W2_mu, W2_rho, b2_mu, b2_rho):
    raise NotImplementedError("write your pallas kernel here")


# speedup vs baseline: 4.6956x; 4.6956x over previous
import functools

import jax
import jax.numpy as jnp
from jax import lax
from jax.experimental import pallas as pl
from jax.experimental.pallas import tpu as pltpu
from jax.experimental.pallas import tpu_sc as plsc

N = 10000
E = 320000
D = 128

NC = 2
NS = 16
NW = NC * NS
EDGES_PER_TILE = E // NW
CHUNK = 80
NCHUNK = EDGES_PER_TILE // CHUNK
ROWS_MAIN = 624
ROWS_TAIL = N - NS * ROWS_MAIN

_mesh = plsc.VectorSubcoreMesh(core_axis_name="c", subcore_axis_name="s")


def _seg_body(x_hbm, src_hbm, dst_hbm, zeros_hbm, out_hbm,
              acc, src_idx, dst_idx, rows, sem):
    c = lax.axis_index("c")
    s = lax.axis_index("s")
    wid = s * NC + c
    base = wid * EDGES_PER_TILE
    row0 = s * ROWS_MAIN

    pltpu.sync_copy(zeros_hbm.at[pl.ds(0, ROWS_MAIN)], acc.at[pl.ds(row0, ROWS_MAIN)])

    @pl.when(s == 0)
    def _():
        pltpu.sync_copy(zeros_hbm.at[pl.ds(0, ROWS_TAIL)],
                        acc.at[pl.ds(NS * ROWS_MAIN, ROWS_TAIL)])

    plsc.subcore_barrier()

    def body(i, carry):
        off = base + i * CHUNK
        pltpu.sync_copy(src_hbm.at[pl.ds(off, CHUNK)], src_idx)
        pltpu.sync_copy(dst_hbm.at[pl.ds(off, CHUNK)], dst_idx)
        pltpu.async_copy(x_hbm.at[src_idx], rows, sem).wait()
        pltpu.sync_copy(rows, acc.at[dst_idx], add=True)
        return carry

    lax.fori_loop(0, NCHUNK, body, 0)
    plsc.subcore_barrier()

    pltpu.sync_copy(acc.at[pl.ds(row0, ROWS_MAIN)],
                    out_hbm.at[c, pl.ds(row0, ROWS_MAIN)])

    @pl.when(s == 0)
    def _():
        pltpu.sync_copy(acc.at[pl.ds(NS * ROWS_MAIN, ROWS_TAIL)],
                        out_hbm.at[c, pl.ds(NS * ROWS_MAIN, ROWS_TAIL)])


_segment_sum_sc = functools.partial(
    pl.kernel,
    out_type=jax.ShapeDtypeStruct((NC, N, D), jnp.float32),
    mesh=_mesh,
    scratch_types=[
        pltpu.VMEM_SHARED((N, D), jnp.float32),
        pltpu.VMEM((CHUNK,), jnp.int32),
        pltpu.VMEM((CHUNK,), jnp.int32),
        pltpu.VMEM((CHUNK, D), jnp.float32),
        pltpu.SemaphoreType.DMA,
    ],
)(_seg_body)


def _dense_tc(p, w_mu, w_rho, eps_w, b_mu, b_rho, eps_b, relu):
    blk = 1000

    def body(p_ref, wmu, wrho, ew, bmu, brho, eb, o_ref):
        w = wmu[...] + jnp.log(1.0 + jnp.exp(wrho[...])) * ew[...]
        b = bmu[...] + jnp.log(1.0 + jnp.exp(brho[...])) * eb[...]
        agg = p_ref[0] + p_ref[1]
        y = jnp.dot(agg, w, preferred_element_type=jnp.float32) + b
        if relu:
            y = jnp.maximum(y, 0.0)
        o_ref[...] = y

    return pl.pallas_call(
        body,
        grid=(N // blk,),
        in_specs=[
            pl.BlockSpec((NC, blk, D), lambda i: (0, i, 0)),
            pl.BlockSpec((D, D), lambda i: (0, 0)),
            pl.BlockSpec((D, D), lambda i: (0, 0)),
            pl.BlockSpec((D, D), lambda i: (0, 0)),
            pl.BlockSpec((1, D), lambda i: (0, 0)),
            pl.BlockSpec((1, D), lambda i: (0, 0)),
            pl.BlockSpec((1, D), lambda i: (0, 0)),
        ],
        out_specs=pl.BlockSpec((blk, D), lambda i: (i, 0)),
        out_shape=jax.ShapeDtypeStruct((N, D), jnp.float32),
    )(p, w_mu, w_rho, eps_w,
      b_mu.reshape(1, D), b_rho.reshape(1, D), eps_b.reshape(1, D))


def kernel(x, edge_index, W1_mu, W1_rho, b1_mu, b1_rho, W2_mu, W2_rho, b2_mu, b2_rho):
    src = edge_index[0]
    dst = edge_index[1]
    zeros = jnp.zeros((ROWS_MAIN, D), jnp.float32)

    k = jax.random.key(42)
    k1, k2 = jax.random.split(k)
    kW1, kb1 = jax.random.split(k1)
    kW2, kb2 = jax.random.split(k2)
    eW1 = jax.random.normal(kW1, (D, D), jnp.float32)
    eb1 = jax.random.normal(kb1, (D,), jnp.float32)
    eW2 = jax.random.normal(kW2, (D, D), jnp.float32)
    eb2 = jax.random.normal(kb2, (D,), jnp.float32)

    p1 = _segment_sum_sc(x, src, dst, zeros)
    h = _dense_tc(p1, W1_mu, W1_rho, eW1, b1_mu, b1_rho, eb1, relu=True)
    p2 = _segment_sum_sc(h, src, dst, zeros)
    out = _dense_tc(p2, W2_mu, W2_rho, eW2, b2_mu, b2_rho, eb2, relu=False)
    return out
